# baseline (speedup 1.0000x reference)
import jax
import jax.numpy as jnp
from jax.experimental import pallas as pl
from jax.experimental.pallas import tpu as pltpu


def _split(x):
    hi = x.astype(jnp.bfloat16)
    lo = (x - hi.astype(jnp.float32)).astype(jnp.bfloat16)
    return hi, lo


def _dot3(ah, al, bh, bl):
    f = jnp.float32
    return (jnp.dot(ah, bl, preferred_element_type=f)
            + jnp.dot(al, bh, preferred_element_type=f)
            + jnp.dot(ah, bh, preferred_element_type=f))


def _node_kernel(Th, Tl, adj_t, X, Z, w1, b1, p1, p2, d2_out, tth_out,
                 ttl_out, acc, hw):
    j = pl.program_id(0)
    nj = pl.num_programs(0)
    BJ = adj_t.shape[1]

    @pl.when(j == 0)
    def _init():
        xh, xl = _split(X[...])
        wh, wl = _split(w1[...])
        hw[...] = _dot3(xh, xl, wh, wl)
        acc[...] = jnp.zeros_like(acc)

    d1 = jnp.sum(Z[...] * p1[...], axis=1, keepdims=True)
    trh = jnp.transpose(Th[pl.ds(j * BJ, BJ), :], (1, 0))
    trl = jnp.transpose(Tl[pl.ds(j * BJ, BJ), :], (1, 0))
    tth_out[...] = trh
    ttl_out[...] = trl
    tr = trh.astype(jnp.float32) + trl.astype(jnp.float32)
    Wh, Wl = _split(tr * d1)
    mult = _dot3(Th[...], Tl[...], Wh, Wl)

    P = mult * adj_t[...]
    trhf = trh.astype(jnp.float32)
    trlf = trl.astype(jnp.float32)
    whf = Wh.astype(jnp.float32)
    wlf = Wl.astype(jnp.float32)
    mdiag = jnp.sum(trhf * whf + trhf * wlf + trlf * whf, axis=0)
    blk = adj_t[pl.ds(j * BJ, BJ), :]
    ii = jax.lax.broadcasted_iota(jnp.int32, (BJ, BJ), 0)
    jj = jax.lax.broadcasted_iota(jnp.int32, (BJ, BJ), 1)
    adjd = jnp.sum(jnp.where(ii == jj, blk, 0.0), axis=0)
    corr = adjd * (1.0 - mdiag)
    inv = 1.0 / (jnp.sum(P, axis=0) + corr + 1e-10)
    V = hw[pl.ds(j * BJ, BJ), :] * inv[:, None]
    ph, plo = _split(P)
    vh, vl = _split(V)
    acc[...] += _dot3(ph, plo, vh, vl)
    acc[pl.ds(j * BJ, BJ), :] += corr[:, None] * V

    @pl.when(j == nj - 1)
    def _fin():
        Xv = jnp.maximum(acc[...] + b1[...], 0.0)
        d2_out[...] = jnp.sum(Xv * p2[...], axis=1, keepdims=True)


def _edge_kernel(Tth, Ttl, Tcol_t, adj_t, Z, w2, b2, d2, o_ref, acc):
    j = pl.program_id(0)
    nj = pl.num_programs(0)
    BJ = adj_t.shape[1]

    @pl.when(j == 0)
    def _init():
        acc[...] = jnp.zeros_like(acc)

    Wh, Wl = _split(Tcol_t[...] * d2[...])
    mult = _dot3(Tth[...], Ttl[...], Wh, Wl)

    P = mult * adj_t[...]
    tch, tcl = _split(Tcol_t[...])
    tchf = tch.astype(jnp.float32)
    tclf = tcl.astype(jnp.float32)
    whf = Wh.astype(jnp.float32)
    wlf = Wl.astype(jnp.float32)
    mdiag = jnp.sum(tchf * whf + tchf * wlf + tclf * whf, axis=0)
    blk = adj_t[pl.ds(j * BJ, BJ), :]
    ii = jax.lax.broadcasted_iota(jnp.int32, (BJ, BJ), 0)
    jj = jax.lax.broadcasted_iota(jnp.int32, (BJ, BJ), 1)
    adjd = jnp.sum(jnp.where(ii == jj, blk, 0.0), axis=0)
    corr = adjd * (1.0 - mdiag)
    inv = 1.0 / (jnp.sum(P, axis=0) + corr + 1e-10)

    Ze_t = jnp.maximum(Z[pl.ds(j * BJ, BJ), :], 0.0)
    HW2_t = jnp.dot(Ze_t, w2[...], preferred_element_type=jnp.float32)
    V = HW2_t * inv[:, None]
    acc[...] += jnp.dot(P, V, preferred_element_type=jnp.float32)
    acc[pl.ds(j * BJ, BJ), :] += corr[:, None] * V

    @pl.when(j == nj - 1)
    def _fin():
        out2 = acc[...] + b2[...]
        m = jnp.max(out2, axis=0, keepdims=True)
        sh = out2 - m
        lse = jnp.log(jnp.sum(jnp.exp(sh), axis=0, keepdims=True))
        o_ref[...] = sh - lse


def kernel(X, Z, adj_e, adj_v, T, w1, b1, p1, w2, b2, p2):
    N, E = T.shape
    NHID = w1.shape[1]
    NCLASS = w2.shape[1]

    T_hi = T.astype(jnp.bfloat16)
    T_lo = (T - T_hi.astype(jnp.float32)).astype(jnp.bfloat16)

    BJ1 = 256
    nj1 = N // BJ1
    d2, Tt_hi, Tt_lo = pl.pallas_call(
        _node_kernel,
        grid=(nj1,),
        in_specs=[
            pl.BlockSpec((N, E), lambda j: (0, 0)),
            pl.BlockSpec((N, E), lambda j: (0, 0)),
            pl.BlockSpec((N, BJ1), lambda j: (0, j)),
            pl.BlockSpec((N, X.shape[1]), lambda j: (0, 0)),
            pl.BlockSpec((E, Z.shape[1]), lambda j: (0, 0)),
            pl.BlockSpec(w1.shape, lambda j: (0, 0)),
            pl.BlockSpec((1, NHID), lambda j: (0, 0)),
            pl.BlockSpec(p1.shape, lambda j: (0, 0)),
            pl.BlockSpec(p2.shape, lambda j: (0, 0)),
        ],
        out_specs=[
            pl.BlockSpec((N, 1), lambda j: (0, 0)),
            pl.BlockSpec((E, BJ1), lambda j: (0, j)),
            pl.BlockSpec((E, BJ1), lambda j: (0, j)),
        ],
        out_shape=[
            jax.ShapeDtypeStruct((N, 1), jnp.float32),
            jax.ShapeDtypeStruct((E, N), jnp.bfloat16),
            jax.ShapeDtypeStruct((E, N), jnp.bfloat16),
        ],
        scratch_shapes=[
            pltpu.VMEM((N, NHID), jnp.float32),
            pltpu.VMEM((N, NHID), jnp.float32),
        ],
        compiler_params=pltpu.CompilerParams(
            dimension_semantics=("arbitrary",)),
    )(T_hi, T_lo, adj_v, X, Z, w1, b1.reshape(1, NHID), p1, p2)

    BJ2 = 256
    nj2 = E // BJ2
    out = pl.pallas_call(
        _edge_kernel,
        grid=(nj2,),
        in_specs=[
            pl.BlockSpec((E, N), lambda j: (0, 0)),
            pl.BlockSpec((E, N), lambda j: (0, 0)),
            pl.BlockSpec((N, BJ2), lambda j: (0, j)),
            pl.BlockSpec((E, BJ2), lambda j: (0, j)),
            pl.BlockSpec((E, Z.shape[1]), lambda j: (0, 0)),
            pl.BlockSpec(w2.shape, lambda j: (0, 0)),
            pl.BlockSpec((1, NCLASS), lambda j: (0, 0)),
            pl.BlockSpec((N, 1), lambda j: (0, 0)),
        ],
        out_specs=pl.BlockSpec((E, NCLASS), lambda j: (0, 0)),
        out_shape=jax.ShapeDtypeStruct((E, NCLASS), jnp.float32),
        scratch_shapes=[pltpu.VMEM((E, NCLASS), jnp.float32)],
        compiler_params=pltpu.CompilerParams(
            dimension_semantics=("arbitrary",)),
    )(Tt_hi, Tt_lo, T, adj_e, Z, w2, b2.reshape(1, NCLASS), d2)
    return out
